# baseline (speedup 1.0000x reference)
import jax
import jax.numpy as jnp
from jax.experimental import pallas as pl
from jax.experimental.pallas import tpu as pltpu

_B = 8


def _pos_kernel(col_ref, row_ref, out_ref, scratch, sem):
    s = pl.program_id(0)
    d, hh, w = scratch.shape[1], scratch.shape[2], scratch.shape[3]

    @pl.when(s == 0)
    def _col():
        colT = col_ref[...].T
        scratch[0] = jnp.broadcast_to(colT[:, None, :], (d, hh, w))

    @pl.when(s == 1)
    def _row():
        rowT = row_ref[...].T
        scratch[1] = jnp.broadcast_to(rowT[:, :, None], (d, hh, w))

    def copies(ss):
        return [
            pltpu.make_async_copy(
                scratch.at[ss],
                out_ref.at[b, pl.ds(ss * d, d), :, :],
                sem.at[ss, b],
            )
            for b in range(_B)
        ]

    for c in copies(s):
        c.start()

    @pl.when(s == 1)
    def _drain():
        for c in copies(1):
            c.wait()
        for c in copies(0):
            c.wait()


def kernel(x, row_embed, col_embed):
    B, C, H, W = x.shape
    d = col_embed.shape[1]

    col = col_embed[:W]
    row = row_embed[:H]

    out = pl.pallas_call(
        _pos_kernel,
        grid=(2,),
        in_specs=[
            pl.BlockSpec((W, d), lambda s: (0, 0)),
            pl.BlockSpec((H, d), lambda s: (0, 0)),
        ],
        out_specs=pl.BlockSpec(memory_space=pltpu.MemorySpace.HBM),
        out_shape=jax.ShapeDtypeStruct((B, C, H, W), x.dtype),
        scratch_shapes=[
            pltpu.VMEM((2, d, H, W), jnp.float32),
            pltpu.SemaphoreType.DMA((2, _B)),
        ],
    )(col, row)
    return out
